# baseline (speedup 1.0000x reference)
import jax
import jax.numpy as jnp
from jax import lax
from jax.experimental import pallas as pl
from jax.experimental.pallas import tpu as pltpu
from jax.experimental.pallas import tpu_sc as plsc

N, D = 1048576, 16
NWORDS = N * D
ROWS, COLS = NWORDS // 128, 128
NFLIPS = 262144
NC, NS, L = 2, 16, 16
SHARD = NWORDS // (NC * NS)
HALF = NWORDS // NC
NCHUNK = 32
CHUNK = SHARD // NCHUNK
LSLICE = NFLIPS // NS
LSTEPS = 8
LCH = LSLICE // LSTEPS
BCAP = 1024
CCAP = 512
CSH = 9
CHSH = 14


def _body(bits, fidx, fbp, out, sidx0, sbp0, sidx1, sbp1, abin, acnt,
          stage0, stage1, bbin, bcnt, lcnt, cnew, data0, data1, data2,
          shbin, shcnt, si0, si1, si2, so0, so1, so2, sl0, sl1, ss0, ss1):
  cid = lax.axis_index("c")
  sid = lax.axis_index("s")
  wid = cid * NS + sid
  base = wid * SHARD
  iota = lax.iota(jnp.int32, L)

  rowtop = pl.multiple_of(cid * 8, 8)

  def cols(c):
    p0 = wid * SHARD + c * CHUNK
    colstart = jnp.left_shift(jnp.bitwise_and(jnp.right_shift(p0, 10), 8191),
                              7)
    return pl.ds(pl.multiple_of(colstart, CHUNK // 8), CHUNK // 8)

  cur_in = [pltpu.async_copy(bits.at[pl.ds(rowtop, 8), cols(0)], data0, si0),
            pltpu.async_copy(bits.at[pl.ds(rowtop, 8), cols(1)], data1, si1),
            pltpu.async_copy(bits.at[pl.ds(rowtop, 8), cols(2)], data2, si2)]

  acnt[...] = jnp.zeros((L,), jnp.int32)

  lbufs = ((sidx0, sbp0), (sidx1, sbp1))
  lsems = (sl0, sl1)

  def list_dma(lc, b):
    off = sid * LSLICE + lc * LCH
    h1 = pltpu.async_copy(fidx.at[pl.ds(off, LCH)], lbufs[b][0], lsems[b])
    h2 = pltpu.async_copy(fbp.at[pl.ds(off, LCH)], lbufs[b][1], lsems[b])
    return (h1, h2)

  pend = [list_dma(0, 0), list_dma(1, 1)]

  for lc in range(LSTEPS):
    b = lc % 2
    for h in pend[b]:
      h.wait()
    sidx, sbp = lbufs[b]

    def vec(i, _, sidx=sidx, sbp=sbp):
      sl = pl.ds(i * L, L)
      jv = sidx[sl]
      n = jnp.right_shift(jv, 4)
      d = jnp.bitwise_and(jv, D - 1)
      phys = jnp.left_shift(jnp.bitwise_and(d, 8), 20) + \
          jnp.left_shift(jnp.right_shift(n, 7), 10) + \
          jnp.left_shift(jnp.bitwise_and(d, 7), 7) + \
          jnp.bitwise_and(n, COLS - 1)
      m = jnp.right_shift(phys, 23) == cid
      b = jnp.bitwise_and(jnp.right_shift(phys, 19), NS - 1)
      rel = jnp.bitwise_and(phys, SHARD - 1)
      packed = rel + jnp.left_shift(sbp[sl], 19)
      cg = plsc.load_gather(acnt, [b], mask=m)
      cnts, lastm = plsc.scan_count(b, m)
      pos = jnp.where(m, jnp.left_shift(b, 10) + cg + cnts - 1, 0)
      plsc.store_scatter(abin, [pos], packed, mask=m)
      plsc.store_scatter(acnt, [b], cg + cnts, mask=m & lastm)
      return 0

    lax.fori_loop(0, LCH // L, vec, 0)
    if lc + 2 < LSTEPS:
      pend[b] = list_dma(lc + 2, b)

  pltpu.sync_copy(abin.at[pl.ds(0, NS * BCAP)], shbin.at[sid])
  pltpu.sync_copy(acnt, shcnt.at[sid])
  plsc.subcore_barrier()

  pltpu.sync_copy(shcnt, lcnt)
  bcnt[pl.ds(0, L)] = jnp.zeros((L,), jnp.int32)
  bcnt[pl.ds(L, L)] = jnp.zeros((L,), jnp.int32)

  sbufs = (stage0, stage1)
  ssems = (ss0, ss1)

  def stage_dma(w, b):
    return pltpu.async_copy(shbin.at[w, pl.ds(sid * BCAP, BCAP)], sbufs[b],
                            ssems[b])

  spend = [stage_dma(0, 0), stage_dma(1, 1)]

  for w in range(NS):
    b2 = w % 2
    spend[b2].wait()
    stage = sbufs[b2]
    nw = plsc.load_gather(
        lcnt, [jnp.full((L,), w, jnp.int32), jnp.full((L,), sid, jnp.int32)]
    )[0]

    def vec(i, _, stage=stage, nw=nw):
      sl = pl.ds(i * L, L)
      lanes = (i * L + iota) < nw
      packed = stage[sl]
      rel = jnp.bitwise_and(packed, SHARD - 1)
      b = jnp.right_shift(rel, CHSH)
      cg = plsc.load_gather(bcnt, [b], mask=lanes)
      cnts, lastm = plsc.scan_count(b, lanes)
      pos = jnp.where(lanes, jnp.left_shift(b, CSH) + cg + cnts - 1, 0)
      plsc.store_scatter(bbin, [pos], packed, mask=lanes)
      plsc.store_scatter(bcnt, [b], cg + cnts, mask=lanes & lastm)
      return 0

    lax.fori_loop(0, (nw + L - 1) // L, vec, 0)
    if w + 2 < NS:
      spend[b2] = stage_dma(w + 2, b2)

  bufs = (data0, data1, data2)
  in_sems = (si0, si1, si2)
  out_sems = (so0, so1, so2)
  prev_out = [None, None, None]

  for c in range(NCHUNK):
    buf = bufs[c % 3]
    cur_in[c % 3].wait()
    if 2 <= c and c + 1 < NCHUNK:
      nb = (c + 1) % 3
      if prev_out[nb] is not None:
        prev_out[nb].wait()
        prev_out[nb] = None
      cur_in[nb] = pltpu.async_copy(
          bits.at[pl.ds(rowtop, 8), cols(c + 1)], bufs[nb], in_sems[nb])

    nc2 = bcnt[pl.ds((c // L) * L, L)][c % L]

    def gat(i, _, buf=buf, c=c, nc2=nc2):
      sl = pl.ds(c * CCAP + i * L, L)
      lanes = (i * L + iota) < nc2
      packed = jnp.where(lanes, bbin[sl], 0)
      relc = jnp.bitwise_and(packed, CHUNK - 1)
      rw = jnp.bitwise_and(jnp.right_shift(relc, 7), 7)
      cl = jnp.left_shift(jnp.right_shift(relc, 10), 7) + \
          jnp.bitwise_and(relc, COLS - 1)
      orig = plsc.bitcast(plsc.load_gather(buf, [rw, cl], mask=lanes),
                          jnp.int32)
      mv = jnp.left_shift(jnp.full((L,), 1, jnp.int32),
                          jnp.right_shift(packed, 19))
      cnew[pl.ds(i * L, L)] = plsc.bitcast(jnp.bitwise_xor(orig, mv),
                                           jnp.float32)
      return 0

    lax.fori_loop(0, (nc2 + L - 1) // L, gat, 0)

    def sca(i, _, buf=buf, c=c, nc2=nc2):
      sl = pl.ds(c * CCAP + i * L, L)
      lanes = (i * L + iota) < nc2
      packed = jnp.where(lanes, bbin[sl], 0)
      relc = jnp.bitwise_and(packed, CHUNK - 1)
      rw = jnp.bitwise_and(jnp.right_shift(relc, 7), 7)
      cl = jnp.left_shift(jnp.right_shift(relc, 10), 7) + \
          jnp.bitwise_and(relc, COLS - 1)
      _, lastm = plsc.scan_count(relc, lanes)
      plsc.store_scatter(buf, [rw, cl], cnew[pl.ds(i * L, L)],
                         mask=lanes & lastm)
      return 0

    lax.fori_loop(0, (nc2 + L - 1) // L, sca, 0)

    prev_out[c % 3] = pltpu.async_copy(buf, out.at[pl.ds(rowtop, 8), cols(c)],
                                       out_sems[c % 3])

  for h in prev_out:
    if h is not None:
      h.wait()


_mesh = plsc.VectorSubcoreMesh(
    core_axis_name="c", subcore_axis_name="s", num_cores=NC, num_subcores=NS
)

_flip = pl.kernel(
    _body,
    out_type=jax.ShapeDtypeStruct((D, N), jnp.float32),
    mesh=_mesh,
    compiler_params=pltpu.CompilerParams(needs_layout_passes=False),
    scratch_types=[
        pltpu.VMEM((LCH,), jnp.int32),
        pltpu.VMEM((LCH,), jnp.int32),
        pltpu.VMEM((LCH,), jnp.int32),
        pltpu.VMEM((LCH,), jnp.int32),
        pltpu.VMEM((NS * BCAP + L,), jnp.int32),
        pltpu.VMEM((L,), jnp.int32),
        pltpu.VMEM((BCAP,), jnp.int32),
        pltpu.VMEM((BCAP,), jnp.int32),
        pltpu.VMEM((NCHUNK * CCAP + L,), jnp.int32),
        pltpu.VMEM((NCHUNK,), jnp.int32),
        pltpu.VMEM((NS, NS), jnp.int32),
        pltpu.VMEM((CCAP + L,), jnp.float32),
        pltpu.VMEM((8, CHUNK // 8), jnp.float32),
        pltpu.VMEM((8, CHUNK // 8), jnp.float32),
        pltpu.VMEM((8, CHUNK // 8), jnp.float32),
        pltpu.VMEM_SHARED((NS, NS * BCAP), jnp.int32),
        pltpu.VMEM_SHARED((NS, NS), jnp.int32),
        pltpu.SemaphoreType.DMA,
        pltpu.SemaphoreType.DMA,
        pltpu.SemaphoreType.DMA,
        pltpu.SemaphoreType.DMA,
        pltpu.SemaphoreType.DMA,
        pltpu.SemaphoreType.DMA,
        pltpu.SemaphoreType.DMA,
        pltpu.SemaphoreType.DMA,
        pltpu.SemaphoreType.DMA,
        pltpu.SemaphoreType.DMA,
    ],
)


@jax.jit
def kernel(input, flip_idx, bit_pos):
  out = _flip(input.T, flip_idx, bit_pos.astype(jnp.int32))
  return out.T


# speedup vs baseline: 1.0912x; 1.0912x over previous
import jax
import jax.numpy as jnp
from jax import lax
from jax.experimental import pallas as pl
from jax.experimental.pallas import tpu as pltpu
from jax.experimental.pallas import tpu_sc as plsc

N, D = 1048576, 16
NWORDS = N * D
ROWS, COLS = NWORDS // 128, 128
NFLIPS = 262144
NC, NS, L = 2, 16, 16
SHARD = NWORDS // (NC * NS)
HALF = NWORDS // NC
NCHUNK = 16
CHUNK = SHARD // NCHUNK
CROWS = CHUNK // COLS
LSLICE = NFLIPS // NS
LSTEPS = 8
LCH = LSLICE // LSTEPS
BCAP = 1024
CCAP = 1024


def _body(bits, fidx, fbp, out, sidx0, sbp0, sidx1, sbp1, abin, acnt,
          stage0, stage1, bbin, bcnt, lcnt, cnew, data0, data1, shbin,
          shcnt, si0, si1, so0, so1, sl0, sl1, ss0, ss1):
  cid = lax.axis_index("c")
  sid = lax.axis_index("s")
  wid = cid * NS + sid
  base = wid * SHARD
  iota = lax.iota(jnp.int32, L)

  rowtop = pl.multiple_of(cid * 8, 8)

  def cols(c):
    p0 = wid * SHARD + c * CHUNK
    colstart = jnp.left_shift(jnp.bitwise_and(jnp.right_shift(p0, 10), 8191),
                              7)
    return pl.ds(pl.multiple_of(colstart, CHUNK // 8), CHUNK // 8)

  cur_in = [pltpu.async_copy(bits.at[pl.ds(rowtop, 8), cols(0)], data0, si0),
            pltpu.async_copy(bits.at[pl.ds(rowtop, 8), cols(1)], data1, si1)]

  acnt[...] = jnp.zeros((L,), jnp.int32)

  lbufs = ((sidx0, sbp0), (sidx1, sbp1))
  lsems = (sl0, sl1)

  def list_dma(lc, b):
    off = sid * LSLICE + lc * LCH
    h1 = pltpu.async_copy(fidx.at[pl.ds(off, LCH)], lbufs[b][0], lsems[b])
    h2 = pltpu.async_copy(fbp.at[pl.ds(off, LCH)], lbufs[b][1], lsems[b])
    return (h1, h2)

  pend = [list_dma(0, 0), list_dma(1, 1)]

  for lc in range(LSTEPS):
    b = lc % 2
    for h in pend[b]:
      h.wait()
    sidx, sbp = lbufs[b]

    def vec(i, _, sidx=sidx, sbp=sbp):
      sl = pl.ds(i * L, L)
      jv = sidx[sl]
      n = jnp.right_shift(jv, 4)
      d = jnp.bitwise_and(jv, D - 1)
      phys = jnp.left_shift(jnp.bitwise_and(d, 8), 20) + \
          jnp.left_shift(jnp.right_shift(n, 7), 10) + \
          jnp.left_shift(jnp.bitwise_and(d, 7), 7) + \
          jnp.bitwise_and(n, COLS - 1)
      m = jnp.right_shift(phys, 23) == cid
      b = jnp.bitwise_and(jnp.right_shift(phys, 19), NS - 1)
      rel = jnp.bitwise_and(phys, SHARD - 1)
      packed = rel + jnp.left_shift(sbp[sl], 19)
      cg = plsc.load_gather(acnt, [b], mask=m)
      cnts, lastm = plsc.scan_count(b, m)
      pos = jnp.where(m, jnp.left_shift(b, 10) + cg + cnts - 1, 0)
      plsc.store_scatter(abin, [pos], packed, mask=m)
      plsc.store_scatter(acnt, [b], cg + cnts, mask=m & lastm)
      return 0

    lax.fori_loop(0, LCH // L, vec, 0)
    if lc + 2 < LSTEPS:
      pend[b] = list_dma(lc + 2, b)

  pltpu.sync_copy(abin.at[pl.ds(0, NS * BCAP)], shbin.at[sid])
  pltpu.sync_copy(acnt, shcnt.at[sid])
  plsc.subcore_barrier()

  pltpu.sync_copy(shcnt, lcnt)
  bcnt[...] = jnp.zeros((L,), jnp.int32)

  sbufs = (stage0, stage1)
  ssems = (ss0, ss1)

  def stage_dma(w, b):
    return pltpu.async_copy(shbin.at[w, pl.ds(sid * BCAP, BCAP)], sbufs[b],
                            ssems[b])

  spend = [stage_dma(0, 0), stage_dma(1, 1)]

  for w in range(NS):
    b2 = w % 2
    spend[b2].wait()
    stage = sbufs[b2]
    nw = plsc.load_gather(
        lcnt, [jnp.full((L,), w, jnp.int32), jnp.full((L,), sid, jnp.int32)]
    )[0]

    def vec(i, _, stage=stage, nw=nw):
      sl = pl.ds(i * L, L)
      lanes = (i * L + iota) < nw
      packed = stage[sl]
      rel = jnp.bitwise_and(packed, SHARD - 1)
      b = jnp.right_shift(rel, 15)
      cg = plsc.load_gather(bcnt, [b], mask=lanes)
      cnts, lastm = plsc.scan_count(b, lanes)
      pos = jnp.where(lanes, jnp.left_shift(b, 10) + cg + cnts - 1, 0)
      plsc.store_scatter(bbin, [pos], packed, mask=lanes)
      plsc.store_scatter(bcnt, [b], cg + cnts, mask=lanes & lastm)
      return 0

    lax.fori_loop(0, (nw + L - 1) // L, vec, 0)
    if w + 2 < NS:
      spend[b2] = stage_dma(w + 2, b2)

  bufs = (data0, data1)
  in_sems = (si0, si1)
  out_sems = (so0, so1)
  prev_out = [None, None]

  for c in range(NCHUNK):
    buf = bufs[c % 2]
    cur_in[c % 2].wait()
    if 0 < c and c + 1 < NCHUNK:
      if prev_out[(c + 1) % 2] is not None:
        prev_out[(c + 1) % 2].wait()
        prev_out[(c + 1) % 2] = None
      cur_in[(c + 1) % 2] = pltpu.async_copy(
          bits.at[pl.ds(rowtop, 8), cols(c + 1)], bufs[(c + 1) % 2],
          in_sems[(c + 1) % 2])

    nc2 = bcnt[...][c]

    def gat(i, _, buf=buf, c=c, nc2=nc2):
      sl = pl.ds(c * CCAP + i * L, L)
      lanes = (i * L + iota) < nc2
      packed = jnp.where(lanes, bbin[sl], 0)
      relc = jnp.bitwise_and(packed, CHUNK - 1)
      rw = jnp.bitwise_and(jnp.right_shift(relc, 7), 7)
      cl = jnp.left_shift(jnp.right_shift(relc, 10), 7) + \
          jnp.bitwise_and(relc, COLS - 1)
      orig = plsc.bitcast(plsc.load_gather(buf, [rw, cl], mask=lanes),
                          jnp.int32)
      mv = jnp.left_shift(jnp.full((L,), 1, jnp.int32),
                          jnp.right_shift(packed, 19))
      cnew[pl.ds(i * L, L)] = plsc.bitcast(jnp.bitwise_xor(orig, mv),
                                           jnp.float32)
      return 0

    lax.fori_loop(0, (nc2 + L - 1) // L, gat, 0)

    def sca(i, _, buf=buf, c=c, nc2=nc2):
      sl = pl.ds(c * CCAP + i * L, L)
      lanes = (i * L + iota) < nc2
      packed = jnp.where(lanes, bbin[sl], 0)
      relc = jnp.bitwise_and(packed, CHUNK - 1)
      rw = jnp.bitwise_and(jnp.right_shift(relc, 7), 7)
      cl = jnp.left_shift(jnp.right_shift(relc, 10), 7) + \
          jnp.bitwise_and(relc, COLS - 1)
      _, lastm = plsc.scan_count(relc, lanes)
      plsc.store_scatter(buf, [rw, cl], cnew[pl.ds(i * L, L)],
                         mask=lanes & lastm)
      return 0

    lax.fori_loop(0, (nc2 + L - 1) // L, sca, 0)

    prev_out[c % 2] = pltpu.async_copy(buf, out.at[pl.ds(rowtop, 8), cols(c)],
                                       out_sems[c % 2])

  for h in prev_out:
    if h is not None:
      h.wait()


_mesh = plsc.VectorSubcoreMesh(
    core_axis_name="c", subcore_axis_name="s", num_cores=NC, num_subcores=NS
)

_flip = pl.kernel(
    _body,
    out_type=jax.ShapeDtypeStruct((D, N), jnp.float32),
    mesh=_mesh,
    compiler_params=pltpu.CompilerParams(needs_layout_passes=False),
    scratch_types=[
        pltpu.VMEM((LCH,), jnp.int32),
        pltpu.VMEM((LCH,), jnp.int32),
        pltpu.VMEM((LCH,), jnp.int32),
        pltpu.VMEM((LCH,), jnp.int32),
        pltpu.VMEM((NS * BCAP + L,), jnp.int32),
        pltpu.VMEM((L,), jnp.int32),
        pltpu.VMEM((BCAP,), jnp.int32),
        pltpu.VMEM((BCAP,), jnp.int32),
        pltpu.VMEM((NCHUNK * CCAP + L,), jnp.int32),
        pltpu.VMEM((L,), jnp.int32),
        pltpu.VMEM((NS, NS), jnp.int32),
        pltpu.VMEM((CCAP + L,), jnp.float32),
        pltpu.VMEM((8, CHUNK // 8), jnp.float32),
        pltpu.VMEM((8, CHUNK // 8), jnp.float32),
        pltpu.VMEM_SHARED((NS, NS * BCAP), jnp.int32),
        pltpu.VMEM_SHARED((NS, NS), jnp.int32),
        pltpu.SemaphoreType.DMA,
        pltpu.SemaphoreType.DMA,
        pltpu.SemaphoreType.DMA,
        pltpu.SemaphoreType.DMA,
        pltpu.SemaphoreType.DMA,
        pltpu.SemaphoreType.DMA,
        pltpu.SemaphoreType.DMA,
        pltpu.SemaphoreType.DMA,
    ],
)


@jax.jit
def kernel(input, flip_idx, bit_pos):
  out = _flip(input.T, flip_idx, bit_pos.astype(jnp.int32))
  return out.T
